# Initial kernel scaffold; baseline (speedup 1.0000x reference)
import functools

import jax
import jax.numpy as jnp
from jax import lax
from jax.experimental import pallas as pl
from jax.experimental.pallas import tpu as pltpu
from jax.experimental.pallas import tpu_sc as plsc

N, E, D_IN, D_HID, D_OUT = 10000, 320000, 128, 256, 128

CHUNK = 128
E_PAD = 323584
NSUB = 16
NCORE = 2
EPS = E_PAD // NSUB
NCH = EPS // CHUNK
EPW = E_PAD // (NSUB * NCORE)
NCH_DEG = EPW // CHUNK

ACC_ROWS = 10016
TRASH = 10008
RPS = ACC_ROWS // NSUB
DEG_ROWS = 10240
DPS = DEG_ROWS // NSUB

_mesh = functools.partial(
    plsc.VectorSubcoreMesh, core_axis_name="c", subcore_axis_name="s")


@functools.partial(
    pl.kernel,
    out_type=jax.ShapeDtypeStruct((NCORE, DEG_ROWS), jnp.float32),
    mesh=_mesh(),
    scratch_types=[
        pltpu.VMEM_SHARED((DEG_ROWS,), jnp.float32),
        pltpu.VMEM((NCH_DEG, CHUNK), jnp.int32),
        pltpu.VMEM((CHUNK,), jnp.float32),
    ],
)
def _sc_degree(dst_hbm, zer_hbm, out_hbm, acc, idx_v, ones_v):
    cid = lax.axis_index("c")
    sid = lax.axis_index("s")
    wid = sid * NCORE + cid
    pltpu.sync_copy(zer_hbm, acc.at[pl.ds(sid * DPS, DPS)])
    pltpu.sync_copy(dst_hbm.at[wid], idx_v)
    for j in range(CHUNK // 16):
        ones_v[pl.ds(j * 16, 16)] = jnp.full((16,), 1.0, jnp.float32)
    plsc.subcore_barrier()

    @pl.loop(0, NCH_DEG)
    def _(i):
        pltpu.sync_copy(ones_v, acc.at[idx_v.at[i]], add=True)

    plsc.subcore_barrier()
    pltpu.sync_copy(acc.at[pl.ds(sid * DPS, DPS)],
                    out_hbm.at[cid, pl.ds(sid * DPS, DPS)])


def _make_sc_aggregate(dh):
    @functools.partial(
        pl.kernel,
        out_type=(jax.ShapeDtypeStruct((ACC_ROWS, dh), jnp.float32),
                  jax.ShapeDtypeStruct((ACC_ROWS, dh), jnp.float32)),
        mesh=_mesh(),
        scratch_types=[
            pltpu.VMEM_SHARED((ACC_ROWS, dh), jnp.float32),
            pltpu.VMEM((NCH, CHUNK), jnp.int32),
            pltpu.VMEM((NCH, CHUNK), jnp.int32),
            pltpu.VMEM((CHUNK, dh), jnp.float32),
            pltpu.SemaphoreType.DMA,
        ],
    )
    def agg(ga_hbm, gb_hbm, src_hbm, dst_hbm, zer_hbm, outa_hbm, outb_hbm,
            acc, src_v, dst_v, rows_v, sem):
        cid = lax.axis_index("c")
        sid = lax.axis_index("s")
        pltpu.sync_copy(zer_hbm, acc.at[pl.ds(sid * RPS, RPS)])
        pltpu.sync_copy(src_hbm.at[sid], src_v)
        pltpu.sync_copy(dst_hbm.at[sid], dst_v)
        plsc.subcore_barrier()

        def half(g_hbm, out_hbm):
            @pl.loop(0, NCH)
            def _(i):
                pltpu.async_copy(g_hbm.at[src_v.at[i]], rows_v, sem).wait()
                pltpu.sync_copy(rows_v, acc.at[dst_v.at[i]], add=True)

            plsc.subcore_barrier()
            pltpu.sync_copy(acc.at[pl.ds(sid * RPS, RPS)],
                            out_hbm.at[pl.ds(sid * RPS, RPS)])

        @pl.when(cid == 0)
        def _():
            half(ga_hbm, outa_hbm)

        @pl.when(cid == 1)
        def _():
            half(gb_hbm, outb_hbm)

    return agg


_sc_agg_hid = _make_sc_aggregate(D_HID // 2)
_sc_agg_out = _make_sc_aggregate(D_OUT // 2)


_BR = 400
_GRID = N // _BR


def _dis_of(d_ref):
    return lax.rsqrt(d_ref[:, 0:1] + d_ref[:, 1:2] + 1.0)


def _tc1_body(d_ref, x_ref, w_ref, ga_ref, gb_ref):
    dis = _dis_of(d_ref)
    g = jnp.dot(x_ref[:], w_ref[:], preferred_element_type=jnp.float32) * dis
    ga_ref[:] = g[:, :D_HID // 2]
    gb_ref[:] = g[:, D_HID // 2:]


def _tc2_body(d_ref, sa_ref, sb_ref, ga_ref, gb_ref, b_ref, w_ref,
              oa_ref, ob_ref):
    dis = _dis_of(d_ref)
    ha = jnp.maximum((sa_ref[:] + ga_ref[:]) * dis + b_ref[0, :D_HID // 2], 0.0)
    hb = jnp.maximum((sb_ref[:] + gb_ref[:]) * dis + b_ref[0, D_HID // 2:], 0.0)
    h = jnp.concatenate([ha, hb], axis=1)
    g = jnp.dot(h, w_ref[:], preferred_element_type=jnp.float32) * dis
    oa_ref[:] = g[:, :D_OUT // 2]
    ob_ref[:] = g[:, D_OUT // 2:]


def _tc3_body(d_ref, sa_ref, sb_ref, ga_ref, gb_ref, b_ref, z_ref):
    dis = _dis_of(d_ref)
    za = (sa_ref[:] + ga_ref[:]) * dis + b_ref[0, :D_OUT // 2]
    zb = (sb_ref[:] + gb_ref[:]) * dis + b_ref[0, D_OUT // 2:]
    z_ref[:] = jnp.concatenate([za, zb], axis=1)


def _row_spec(c):
    return pl.BlockSpec((_BR, c), lambda i: (i, 0))


def _full_spec(r, c):
    return pl.BlockSpec((r, c), lambda i: (0, 0))


_tc1 = pl.pallas_call(
    _tc1_body,
    grid=(_GRID,),
    in_specs=[_row_spec(2), _row_spec(D_IN), _full_spec(D_IN, D_HID)],
    out_specs=[_row_spec(D_HID // 2)] * 2,
    out_shape=[jax.ShapeDtypeStruct((N, D_HID // 2), jnp.float32)] * 2,
)

_tc2 = pl.pallas_call(
    _tc2_body,
    grid=(_GRID,),
    in_specs=[_row_spec(2)] + [_row_spec(D_HID // 2)] * 4
    + [_full_spec(1, D_HID), _full_spec(D_HID, D_OUT)],
    out_specs=[_row_spec(D_OUT // 2)] * 2,
    out_shape=[jax.ShapeDtypeStruct((N, D_OUT // 2), jnp.float32)] * 2,
)

_tc3 = pl.pallas_call(
    _tc3_body,
    grid=(_GRID,),
    in_specs=[_row_spec(2)] + [_row_spec(D_OUT // 2)] * 4
    + [_full_spec(1, D_OUT)],
    out_specs=_row_spec(D_OUT),
    out_shape=jax.ShapeDtypeStruct((N, D_OUT), jnp.float32),
)


def kernel(x, edge_index, W1, b1, W2, b2):
    src = edge_index[0].astype(jnp.int32)
    dst = edge_index[1].astype(jnp.int32)
    pad = E_PAD - E
    src_p = jnp.concatenate([src, jnp.zeros((pad,), jnp.int32)])
    dst_p = jnp.concatenate([dst, jnp.full((pad,), TRASH, jnp.int32)])
    src3 = src_p.reshape(NSUB, NCH, CHUNK)
    dst3 = dst_p.reshape(NSUB, NCH, CHUNK)
    dst_deg = dst_p.reshape(NSUB * NCORE, NCH_DEG, CHUNK)

    zer_deg = jnp.zeros((DPS,), jnp.float32)
    zer_hid = jnp.zeros((RPS, D_HID // 2), jnp.float32)
    zer_out = jnp.zeros((RPS, D_OUT // 2), jnp.float32)

    deg2 = _sc_degree(dst_deg, zer_deg)
    dpair = deg2.T[:N]

    g1a, g1b = _tc1(dpair, x, W1)
    s1a, s1b = _sc_agg_hid(g1a, g1b, src3, dst3, zer_hid)
    g2a, g2b = _tc2(dpair, s1a[:N], s1b[:N], g1a, g1b,
                    b1.reshape(1, D_HID), W2)
    s2a, s2b = _sc_agg_out(g2a, g2b, src3, dst3, zer_out)
    z = _tc3(dpair, s2a[:N], s2b[:N], g2a, g2b, b2.reshape(1, D_OUT))
    return z


# speedup vs baseline: 7.7694x; 7.7694x over previous
import functools

import jax
import jax.numpy as jnp
from jax import lax
from jax.experimental import pallas as pl
from jax.experimental.pallas import tpu as pltpu
from jax.experimental.pallas import tpu_sc as plsc

N, E, D_IN, D_HID, D_OUT = 10000, 320000, 128, 256, 128

CHUNK = 128
E_PAD = 327680
NSUB = 16
NCORE = 2
EPS = E_PAD // NSUB
NCH = EPS // CHUNK
GB = 8
NGRP = NCH // GB
EPW = E_PAD // (NSUB * NCORE)
NCH_DEG = EPW // CHUNK

ACC_ROWS = 10112
TRASH = 10008
RPS = ACC_ROWS // NSUB
DEG_ROWS = 10240
DPS = DEG_ROWS // NSUB

_mesh = functools.partial(
    plsc.VectorSubcoreMesh, core_axis_name="c", subcore_axis_name="s")


@functools.partial(
    pl.kernel,
    out_type=jax.ShapeDtypeStruct((NCORE, DEG_ROWS), jnp.float32),
    mesh=_mesh(),
    scratch_types=[
        pltpu.VMEM_SHARED((DEG_ROWS,), jnp.float32),
        pltpu.VMEM((NCH_DEG, CHUNK), jnp.int32),
        pltpu.VMEM((CHUNK,), jnp.float32),
    ],
)
def _sc_degree(dst_hbm, zer_hbm, out_hbm, acc, idx_v, ones_v):
    cid = lax.axis_index("c")
    sid = lax.axis_index("s")
    wid = sid * NCORE + cid
    pltpu.sync_copy(zer_hbm, acc.at[pl.ds(sid * DPS, DPS)])
    pltpu.sync_copy(dst_hbm.at[wid], idx_v)
    for j in range(CHUNK // 16):
        ones_v[pl.ds(j * 16, 16)] = jnp.full((16,), 1.0, jnp.float32)
    plsc.subcore_barrier()

    @pl.loop(0, NCH_DEG)
    def _(i):
        pltpu.sync_copy(ones_v, acc.at[idx_v.at[i]], add=True)

    plsc.subcore_barrier()
    pltpu.sync_copy(acc.at[pl.ds(sid * DPS, DPS)],
                    out_hbm.at[cid, pl.ds(sid * DPS, DPS)])


def _make_sc_aggregate(dh):
    @functools.partial(
        pl.kernel,
        out_type=(jax.ShapeDtypeStruct((ACC_ROWS, dh), jnp.float32),
                  jax.ShapeDtypeStruct((ACC_ROWS, dh), jnp.float32)),
        mesh=_mesh(),
        scratch_types=[
            pltpu.VMEM_SHARED((ACC_ROWS, dh), jnp.float32),
            pltpu.VMEM((GB, CHUNK), jnp.int32),
            pltpu.VMEM((GB, CHUNK), jnp.int32),
            pltpu.VMEM((CHUNK, dh), jnp.float32),
            pltpu.SemaphoreType.DMA,
        ],
    )
    def agg(ga_hbm, gb_hbm, src_hbm, dst_hbm, zer_hbm, outa_hbm, outb_hbm,
            acc, src_v, dst_v, rows_v, sem):
        cid = lax.axis_index("c")
        sid = lax.axis_index("s")
        pltpu.sync_copy(zer_hbm, acc.at[pl.ds(sid * RPS, RPS)])
        plsc.subcore_barrier()

        def half(g_hbm, out_hbm):
            @pl.loop(0, NGRP)
            def _(g):
                pltpu.sync_copy(src_hbm.at[sid, pl.ds(g * GB, GB)], src_v)
                pltpu.sync_copy(dst_hbm.at[sid, pl.ds(g * GB, GB)], dst_v)
                for j in range(GB):
                    pltpu.async_copy(g_hbm.at[src_v.at[j]], rows_v, sem).wait()
                    pltpu.sync_copy(rows_v, acc.at[dst_v.at[j]], add=True)

            plsc.subcore_barrier()
            pltpu.sync_copy(acc.at[pl.ds(sid * RPS, RPS)],
                            out_hbm.at[pl.ds(sid * RPS, RPS)])

        @pl.when(cid == 0)
        def _():
            half(ga_hbm, outa_hbm)

        @pl.when(cid == 1)
        def _():
            half(gb_hbm, outb_hbm)

    return agg


_sc_agg_hid = _make_sc_aggregate(D_HID // 2)

NGRP2 = NCH_DEG // GB


@functools.partial(
    pl.kernel,
    out_type=(jax.ShapeDtypeStruct((ACC_ROWS, D_OUT), jnp.float32),
              jax.ShapeDtypeStruct((ACC_ROWS, D_OUT), jnp.float32)),
    mesh=_mesh(),
    scratch_types=[
        pltpu.VMEM_SHARED((ACC_ROWS, D_OUT), jnp.float32),
        pltpu.VMEM((GB, CHUNK), jnp.int32),
        pltpu.VMEM((GB, CHUNK), jnp.int32),
        pltpu.VMEM((CHUNK, D_OUT), jnp.float32),
        pltpu.SemaphoreType.DMA,
    ],
)
def _sc_agg_out(g_hbm, src_hbm, dst_hbm, zer_hbm, out0_hbm, out1_hbm,
                acc, src_v, dst_v, rows_v, sem):
    cid = lax.axis_index("c")
    sid = lax.axis_index("s")
    wid = sid * NCORE + cid
    pltpu.sync_copy(zer_hbm, acc.at[pl.ds(sid * RPS, RPS)])
    plsc.subcore_barrier()

    @pl.loop(0, NGRP2)
    def _(g):
        pltpu.sync_copy(src_hbm.at[wid, pl.ds(g * GB, GB)], src_v)
        pltpu.sync_copy(dst_hbm.at[wid, pl.ds(g * GB, GB)], dst_v)
        for j in range(GB):
            pltpu.async_copy(g_hbm.at[src_v.at[j]], rows_v, sem).wait()
            pltpu.sync_copy(rows_v, acc.at[dst_v.at[j]], add=True)

    plsc.subcore_barrier()

    @pl.when(cid == 0)
    def _():
        pltpu.sync_copy(acc.at[pl.ds(sid * RPS, RPS)],
                        out0_hbm.at[pl.ds(sid * RPS, RPS)])

    @pl.when(cid == 1)
    def _():
        pltpu.sync_copy(acc.at[pl.ds(sid * RPS, RPS)],
                        out1_hbm.at[pl.ds(sid * RPS, RPS)])


_BR = 400
_GRID = N // _BR


def _dis_of(d_ref):
    return lax.rsqrt(d_ref[:, 0:1] + d_ref[:, 1:2] + 1.0)


def _tc1_body(d_ref, x_ref, w_ref, ga_ref, gb_ref):
    dis = _dis_of(d_ref)
    g = jnp.dot(x_ref[:], w_ref[:], preferred_element_type=jnp.float32) * dis
    ga_ref[:] = g[:, :D_HID // 2]
    gb_ref[:] = g[:, D_HID // 2:]


def _tc2_body(d_ref, sa_ref, sb_ref, ga_ref, gb_ref, b_ref, w_ref, o_ref):
    dis = _dis_of(d_ref)
    ha = jnp.maximum((sa_ref[:] + ga_ref[:]) * dis + b_ref[0, :D_HID // 2], 0.0)
    hb = jnp.maximum((sb_ref[:] + gb_ref[:]) * dis + b_ref[0, D_HID // 2:], 0.0)
    h = jnp.concatenate([ha, hb], axis=1)
    o_ref[:] = jnp.dot(h, w_ref[:], preferred_element_type=jnp.float32) * dis


def _tc3_body(d_ref, s0_ref, s1_ref, g_ref, b_ref, z_ref):
    dis = _dis_of(d_ref)
    z_ref[:] = (s0_ref[:] + s1_ref[:] + g_ref[:]) * dis + b_ref[0, :]


def _row_spec(c):
    return pl.BlockSpec((_BR, c), lambda i: (i, 0))


def _full_spec(r, c):
    return pl.BlockSpec((r, c), lambda i: (0, 0))


_tc1 = pl.pallas_call(
    _tc1_body,
    grid=(_GRID,),
    in_specs=[_row_spec(2), _row_spec(D_IN), _full_spec(D_IN, D_HID)],
    out_specs=[_row_spec(D_HID // 2)] * 2,
    out_shape=[jax.ShapeDtypeStruct((N, D_HID // 2), jnp.float32)] * 2,
)

_tc2 = pl.pallas_call(
    _tc2_body,
    grid=(_GRID,),
    in_specs=[_row_spec(2)] + [_row_spec(D_HID // 2)] * 4
    + [_full_spec(1, D_HID), _full_spec(D_HID, D_OUT)],
    out_specs=_row_spec(D_OUT),
    out_shape=jax.ShapeDtypeStruct((N, D_OUT), jnp.float32),
)

_tc3 = pl.pallas_call(
    _tc3_body,
    grid=(_GRID,),
    in_specs=[_row_spec(2)] + [_row_spec(D_OUT)] * 3
    + [_full_spec(1, D_OUT)],
    out_specs=_row_spec(D_OUT),
    out_shape=jax.ShapeDtypeStruct((N, D_OUT), jnp.float32),
)


def kernel(x, edge_index, W1, b1, W2, b2):
    src = edge_index[0].astype(jnp.int32)
    dst = edge_index[1].astype(jnp.int32)
    pad = E_PAD - E
    src_p = jnp.concatenate([src, jnp.zeros((pad,), jnp.int32)])
    dst_p = jnp.concatenate([dst, jnp.full((pad,), TRASH, jnp.int32)])
    src3 = src_p.reshape(NSUB, NCH, CHUNK)
    dst3 = dst_p.reshape(NSUB, NCH, CHUNK)
    srcw = src_p.reshape(NSUB * NCORE, NCH_DEG, CHUNK)
    dstw = dst_p.reshape(NSUB * NCORE, NCH_DEG, CHUNK)

    zer_deg = jnp.zeros((DPS,), jnp.float32)
    zer_hid = jnp.zeros((RPS, D_HID // 2), jnp.float32)
    zer_out = jnp.zeros((RPS, D_OUT), jnp.float32)

    deg2 = _sc_degree(dstw, zer_deg)
    dpair = deg2.T[:N]

    g1a, g1b = _tc1(dpair, x, W1)
    s1a, s1b = _sc_agg_hid(g1a, g1b, src3, dst3, zer_hid)
    g2 = _tc2(dpair, s1a[:N], s1b[:N], g1a, g1b, b1.reshape(1, D_HID), W2)
    s20, s21 = _sc_agg_out(g2, srcw, dstw, zer_out)
    z = _tc3(dpair, s20[:N], s21[:N], g2, b2.reshape(1, D_OUT))
    return z


# speedup vs baseline: 8.3363x; 1.0730x over previous
import functools

import jax
import jax.numpy as jnp
from jax import lax
from jax.experimental import pallas as pl
from jax.experimental.pallas import tpu as pltpu
from jax.experimental.pallas import tpu_sc as plsc

N, E, D_IN, D_HID, D_OUT = 10000, 320000, 128, 256, 128

CHUNK = 128
E_PAD = 327680
NSUB = 16
NCORE = 2
EPS = E_PAD // NSUB
NCH = EPS // CHUNK
GB = 8
NGRP = NCH // GB
EPW = E_PAD // (NSUB * NCORE)
NCH_DEG = EPW // CHUNK

ACC_ROWS = 10112
TRASH = 10008
RPS = ACC_ROWS // NSUB
DEG_ROWS = 10240
DPS = DEG_ROWS // NSUB

_mesh = functools.partial(
    plsc.VectorSubcoreMesh, core_axis_name="c", subcore_axis_name="s")


@functools.partial(
    pl.kernel,
    out_type=jax.ShapeDtypeStruct((NCORE, DEG_ROWS), jnp.float32),
    mesh=_mesh(),
    scratch_types=[
        pltpu.VMEM_SHARED((DEG_ROWS,), jnp.float32),
        pltpu.VMEM((NCH_DEG, CHUNK), jnp.int32),
        pltpu.VMEM((CHUNK,), jnp.float32),
    ],
)
def _sc_degree(dst_hbm, zer_hbm, out_hbm, acc, idx_v, ones_v):
    cid = lax.axis_index("c")
    sid = lax.axis_index("s")
    wid = sid * NCORE + cid
    pltpu.sync_copy(zer_hbm, acc.at[pl.ds(sid * DPS, DPS)])
    pltpu.sync_copy(dst_hbm.at[wid], idx_v)
    for j in range(CHUNK // 16):
        ones_v[pl.ds(j * 16, 16)] = jnp.full((16,), 1.0, jnp.float32)
    plsc.subcore_barrier()

    @pl.loop(0, NCH_DEG)
    def _(i):
        pltpu.sync_copy(ones_v, acc.at[idx_v.at[i]], add=True)

    plsc.subcore_barrier()
    pltpu.sync_copy(acc.at[pl.ds(sid * DPS, DPS)],
                    out_hbm.at[cid, pl.ds(sid * DPS, DPS)])


def _make_sc_aggregate(dh):
    @functools.partial(
        pl.kernel,
        out_type=(jax.ShapeDtypeStruct((ACC_ROWS, dh), jnp.float32),
                  jax.ShapeDtypeStruct((ACC_ROWS, dh), jnp.float32)),
        mesh=_mesh(),
        scratch_types=[
            pltpu.VMEM_SHARED((ACC_ROWS, dh), jnp.float32),
            pltpu.VMEM((GB, CHUNK), jnp.int32),
            pltpu.VMEM((GB, CHUNK), jnp.int32),
            pltpu.VMEM((CHUNK, dh), jnp.float32),
            pltpu.VMEM((CHUNK, dh), jnp.float32),
            pltpu.SemaphoreType.DMA,
            pltpu.SemaphoreType.DMA,
        ],
    )
    def agg(ga_hbm, gb_hbm, src_hbm, dst_hbm, zer_hbm, outa_hbm, outb_hbm,
            acc, src_v, dst_v, rows0, rows1, gsem, ssem):
        cid = lax.axis_index("c")
        sid = lax.axis_index("s")
        pltpu.sync_copy(zer_hbm, acc.at[pl.ds(sid * RPS, RPS)])
        plsc.subcore_barrier()

        def half(g_hbm, out_hbm):
            @pl.loop(0, NGRP)
            def _(g):
                pltpu.sync_copy(src_hbm.at[sid, pl.ds(g * GB, GB)], src_v)
                pltpu.sync_copy(dst_hbm.at[sid, pl.ds(g * GB, GB)], dst_v)
                bufs = (rows0, rows1)
                gcp = [None] * GB
                scp = [None] * GB
                gcp[0] = pltpu.async_copy(g_hbm.at[src_v.at[0]], bufs[0], gsem)
                for j in range(GB):
                    b = bufs[j % 2]
                    gcp[j].wait()
                    scp[j] = pltpu.async_copy(
                        b, acc.at[dst_v.at[j]], ssem, add=True)
                    if j + 1 < GB:
                        if j >= 1:
                            scp[j - 1].wait()
                        gcp[j + 1] = pltpu.async_copy(
                            g_hbm.at[src_v.at[j + 1]], bufs[(j + 1) % 2], gsem)
                scp[GB - 2].wait()
                scp[GB - 1].wait()

            plsc.subcore_barrier()
            pltpu.sync_copy(acc.at[pl.ds(sid * RPS, RPS)],
                            out_hbm.at[pl.ds(sid * RPS, RPS)])

        @pl.when(cid == 0)
        def _():
            half(ga_hbm, outa_hbm)

        @pl.when(cid == 1)
        def _():
            half(gb_hbm, outb_hbm)

    return agg


_sc_agg_hid = _make_sc_aggregate(D_HID // 2)

NGRP2 = NCH_DEG // GB


@functools.partial(
    pl.kernel,
    out_type=(jax.ShapeDtypeStruct((ACC_ROWS, D_OUT), jnp.float32),
              jax.ShapeDtypeStruct((ACC_ROWS, D_OUT), jnp.float32)),
    mesh=_mesh(),
    scratch_types=[
        pltpu.VMEM_SHARED((ACC_ROWS, D_OUT), jnp.float32),
        pltpu.VMEM((GB, CHUNK), jnp.int32),
        pltpu.VMEM((GB, CHUNK), jnp.int32),
        pltpu.VMEM((CHUNK, D_OUT), jnp.float32),
        pltpu.VMEM((CHUNK, D_OUT), jnp.float32),
        pltpu.SemaphoreType.DMA,
        pltpu.SemaphoreType.DMA,
    ],
)
def _sc_agg_out(g_hbm, src_hbm, dst_hbm, zer_hbm, out0_hbm, out1_hbm,
                acc, src_v, dst_v, rows0, rows1, gsem, ssem):
    cid = lax.axis_index("c")
    sid = lax.axis_index("s")
    wid = sid * NCORE + cid
    pltpu.sync_copy(zer_hbm, acc.at[pl.ds(sid * RPS, RPS)])
    plsc.subcore_barrier()

    @pl.loop(0, NGRP2)
    def _(g):
        pltpu.sync_copy(src_hbm.at[wid, pl.ds(g * GB, GB)], src_v)
        pltpu.sync_copy(dst_hbm.at[wid, pl.ds(g * GB, GB)], dst_v)
        bufs = (rows0, rows1)
        gcp = [None] * GB
        scp = [None] * GB
        gcp[0] = pltpu.async_copy(g_hbm.at[src_v.at[0]], bufs[0], gsem)
        for j in range(GB):
            b = bufs[j % 2]
            gcp[j].wait()
            scp[j] = pltpu.async_copy(b, acc.at[dst_v.at[j]], ssem, add=True)
            if j + 1 < GB:
                if j >= 1:
                    scp[j - 1].wait()
                gcp[j + 1] = pltpu.async_copy(
                    g_hbm.at[src_v.at[j + 1]], bufs[(j + 1) % 2], gsem)
        scp[GB - 2].wait()
        scp[GB - 1].wait()

    plsc.subcore_barrier()

    @pl.when(cid == 0)
    def _():
        pltpu.sync_copy(acc.at[pl.ds(sid * RPS, RPS)],
                        out0_hbm.at[pl.ds(sid * RPS, RPS)])

    @pl.when(cid == 1)
    def _():
        pltpu.sync_copy(acc.at[pl.ds(sid * RPS, RPS)],
                        out1_hbm.at[pl.ds(sid * RPS, RPS)])


_BR = 400
_GRID = N // _BR


def _dis_of(d_ref):
    return lax.rsqrt(d_ref[:, 0:1] + d_ref[:, 1:2] + 1.0)


def _tc1_body(d_ref, x_ref, w_ref, ga_ref, gb_ref):
    dis = _dis_of(d_ref)
    g = jnp.dot(x_ref[:], w_ref[:], preferred_element_type=jnp.float32) * dis
    ga_ref[:] = g[:, :D_HID // 2]
    gb_ref[:] = g[:, D_HID // 2:]


def _tc2_body(d_ref, sa_ref, sb_ref, ga_ref, gb_ref, b_ref, w_ref, o_ref):
    dis = _dis_of(d_ref)
    ha = jnp.maximum((sa_ref[:] + ga_ref[:]) * dis + b_ref[0, :D_HID // 2], 0.0)
    hb = jnp.maximum((sb_ref[:] + gb_ref[:]) * dis + b_ref[0, D_HID // 2:], 0.0)
    h = jnp.concatenate([ha, hb], axis=1)
    o_ref[:] = jnp.dot(h, w_ref[:], preferred_element_type=jnp.float32) * dis


def _tc3_body(d_ref, s0_ref, s1_ref, g_ref, b_ref, z_ref):
    dis = _dis_of(d_ref)
    z_ref[:] = (s0_ref[:] + s1_ref[:] + g_ref[:]) * dis + b_ref[0, :]


def _row_spec(c):
    return pl.BlockSpec((_BR, c), lambda i: (i, 0))


def _full_spec(r, c):
    return pl.BlockSpec((r, c), lambda i: (0, 0))


_tc1 = pl.pallas_call(
    _tc1_body,
    grid=(_GRID,),
    in_specs=[_row_spec(2), _row_spec(D_IN), _full_spec(D_IN, D_HID)],
    out_specs=[_row_spec(D_HID // 2)] * 2,
    out_shape=[jax.ShapeDtypeStruct((N, D_HID // 2), jnp.float32)] * 2,
)

_tc2 = pl.pallas_call(
    _tc2_body,
    grid=(_GRID,),
    in_specs=[_row_spec(2)] + [_row_spec(D_HID // 2)] * 4
    + [_full_spec(1, D_HID), _full_spec(D_HID, D_OUT)],
    out_specs=_row_spec(D_OUT),
    out_shape=jax.ShapeDtypeStruct((N, D_OUT), jnp.float32),
)

_tc3 = pl.pallas_call(
    _tc3_body,
    grid=(_GRID,),
    in_specs=[_row_spec(2)] + [_row_spec(D_OUT)] * 3
    + [_full_spec(1, D_OUT)],
    out_specs=_row_spec(D_OUT),
    out_shape=jax.ShapeDtypeStruct((N, D_OUT), jnp.float32),
)


def kernel(x, edge_index, W1, b1, W2, b2):
    src = edge_index[0].astype(jnp.int32)
    dst = edge_index[1].astype(jnp.int32)
    pad = E_PAD - E
    src_p = jnp.concatenate([src, jnp.zeros((pad,), jnp.int32)])
    dst_p = jnp.concatenate([dst, jnp.full((pad,), TRASH, jnp.int32)])
    src3 = src_p.reshape(NSUB, NCH, CHUNK)
    dst3 = dst_p.reshape(NSUB, NCH, CHUNK)
    srcw = src_p.reshape(NSUB * NCORE, NCH_DEG, CHUNK)
    dstw = dst_p.reshape(NSUB * NCORE, NCH_DEG, CHUNK)

    zer_deg = jnp.zeros((DPS,), jnp.float32)
    zer_hid = jnp.zeros((RPS, D_HID // 2), jnp.float32)
    zer_out = jnp.zeros((RPS, D_OUT), jnp.float32)

    deg2 = _sc_degree(dstw, zer_deg)
    dpair = deg2.T[:N]

    g1a, g1b = _tc1(dpair, x, W1)
    s1a, s1b = _sc_agg_hid(g1a, g1b, src3, dst3, zer_hid)
    g2 = _tc2(dpair, s1a[:N], s1b[:N], g1a, g1b, b1.reshape(1, D_HID), W2)
    s20, s21 = _sc_agg_out(g2, srcw, dstw, zer_out)
    z = _tc3(dpair, s20[:N], s21[:N], g2, b2.reshape(1, D_OUT))
    return z
